# baseline (speedup 1.0000x reference)
import functools

import jax
import jax.numpy as jnp
from jax import lax
from jax.experimental import pallas as pl
from jax.experimental.pallas import tpu as pltpu
from jax.experimental.pallas import tpu_sc as plsc

VOCAB = 1000000
HID = 64
B = 16384
L = 50
POS_ROWS = 128

NC = 2
NS = 16
NW = NC * NS

PHR_PW = B // NW
PCHUNK = 16
NCHUNK = PHR_PW // PCHUNK
VECS = HID // 16
POS_COPY = 56


def _sc_body(idx_hbm, w_hbm, pos_hbm, out_hbm, idx_v, buf, pos_v,
             gsem0, gsem1, osem0, osem1):
    wid = lax.axis_index("s") * NC + lax.axis_index("c")
    pltpu.sync_copy(pos_hbm.at[pl.ds(0, POS_COPY)], pos_v)
    gsems = (gsem0, gsem1)
    osems = (osem0, osem1)

    def ph0_of(c):
        return pl.multiple_of(wid * PHR_PW + c * PCHUNK, PCHUNK)

    def load_idx(c, s):
        pltpu.sync_copy(idx_hbm.at[pl.ds(ph0_of(c), PCHUNK)], idx_v.at[s])

    def gather_copies(s):
        return [pltpu.make_async_copy(
                    w_hbm.at[idx_v.at[s].at[j]],
                    buf.at[s].at[j], gsems[s])
                for j in range(PCHUNK)]

    def fire_gather(s):
        for cp in gather_copies(s):
            cp.start()

    def wait_gather(s):
        for cp in gather_copies(s):
            cp.wait()

    def fire_store(c, s):
        pltpu.async_copy(buf.at[s], out_hbm.at[pl.ds(ph0_of(c), PCHUNK)],
                         osems[s])

    def wait_store(s):
        pltpu.make_async_copy(
            buf.at[s], out_hbm.at[pl.ds(0, PCHUNK)], osems[s]).wait()

    def pos_add(s):
        def pos_body(p, carry):
            for q in range(VECS):
                pv = pos_v[p, pl.ds(q * 16, 16)]
                for r in range(PCHUNK):
                    buf[s, r, p, pl.ds(q * 16, 16)] = (
                        buf[s, r, p, pl.ds(q * 16, 16)] + pv)
            return carry
        lax.fori_loop(0, L, pos_body, 0)

    load_idx(0, 0)
    fire_gather(0)

    @pl.loop(0, NCHUNK, step=2)
    def _chunks(c0):
        for b in range(2):
            c = c0 + b
            nxt = c + 1

            @pl.when(nxt < NCHUNK)
            def _prefetch():
                load_idx(nxt, 1 - b)

                @pl.when(c >= 1)
                def _drain_prev_store():
                    wait_store(1 - b)

                fire_gather(1 - b)

            wait_gather(b)
            pos_add(b)
            fire_store(c, b)

    wait_store(0)
    wait_store(1)


@jax.jit
def _phrase_embedding_sc(idx, w, pos):
    mesh = plsc.VectorSubcoreMesh(
        core_axis_name="c", subcore_axis_name="s",
        num_cores=NC, num_subcores=NS)
    call = functools.partial(
        pl.kernel,
        out_type=jax.ShapeDtypeStruct((B, L, HID), jnp.float32),
        mesh=mesh,
        scratch_types=[
            pltpu.VMEM((2, PCHUNK, L), jnp.int32),
            pltpu.VMEM((2, PCHUNK, L, HID), jnp.float32),
            pltpu.VMEM((POS_COPY, HID), jnp.float32),
            pltpu.SemaphoreType.DMA,
            pltpu.SemaphoreType.DMA,
            pltpu.SemaphoreType.DMA,
            pltpu.SemaphoreType.DMA,
        ],
        compiler_params=pltpu.CompilerParams(use_tc_tiling_on_sc=False),
    )(_sc_body)
    return call(idx, w, pos)


def kernel(phrase, W, pos_emb):
    return _phrase_embedding_sc(phrase.astype(jnp.int32), W, pos_emb)
